# baseline (speedup 1.0000x reference)
import jax
import jax.numpy as jnp
from jax import lax
from jax.experimental import pallas as pl
from jax.experimental.pallas import tpu as pltpu
from jax.experimental.pallas import tpu_sc as plsc

N = 10000
E = 320000
D = 64
DP = 128
NODE_DIM = 128
POS_DIM = 3

NUM_CORES = 2
NUM_SUBCORES = 16
NTILES = NUM_CORES * NUM_SUBCORES
CH = 128
NCH = 80
GCH = 16
NG = NCH // GCH
EPT = CH * NCH
EPAD = NTILES * EPT
NPAD = 10112
ZROWS = NPAD // NUM_SUBCORES
LAST_ROWS = N - 15 * ZROWS

_MESH = plsc.VectorSubcoreMesh(
    core_axis_name="c", subcore_axis_name="s",
    num_cores=NUM_CORES, num_subcores=NUM_SUBCORES)


def _sc_body(h_hbm, src_hbm, dst_hbm, z_hbm, agg_out,
             src_v, dst_v, rows0_v, rows1_v, agg_sh, sg0, sg1):
  c = lax.axis_index("c")
  s = lax.axis_index("s")
  wid = s * NUM_CORES + c

  zsl = pl.ds(s * ZROWS, ZROWS)
  pltpu.sync_copy(z_hbm.at[zsl], agg_sh.at[zsl])
  plsc.subcore_barrier()

  del rows1_v, sg1
  pltpu.sync_copy(src_hbm.at[wid], src_v)
  pltpu.sync_copy(dst_hbm.at[wid], dst_v)

  def chunk(i, carry):
    pltpu.async_copy(h_hbm.at[src_v.at[i]], rows0_v, sg0).wait()
    pltpu.sync_copy(rows0_v, agg_sh.at[dst_v.at[i]], add=True)
    return carry
  lax.fori_loop(0, NCH, chunk, 0)

  plsc.subcore_barrier()

  @pl.when(s < NUM_SUBCORES - 1)
  def _():
    osl = pl.ds(s * ZROWS, ZROWS)
    pltpu.sync_copy(agg_sh.at[osl], agg_out.at[c, osl])

  @pl.when(s == NUM_SUBCORES - 1)
  def _():
    osl = pl.ds(s * ZROWS, LAST_ROWS)
    pltpu.sync_copy(agg_sh.at[osl], agg_out.at[c, osl])


_sc_agg = pl.kernel(
    _sc_body,
    out_type=jax.ShapeDtypeStruct((NUM_CORES, N, DP), jnp.float32),
    mesh=_MESH,
    scratch_types=[
        pltpu.VMEM((NCH, CH), jnp.int32),
        pltpu.VMEM((NCH, CH), jnp.int32),
        pltpu.VMEM((CH, DP), jnp.float32),
        pltpu.VMEM((CH, DP), jnp.float32),
        pltpu.VMEM_SHARED((NPAD, DP), jnp.float32),
        pltpu.SemaphoreType.DMA,
        pltpu.SemaphoreType.DMA,
    ],
)


def _pad_cols(vals):
  r = vals.shape[0]
  return jnp.concatenate(
      [vals, jnp.ones((r, 1), jnp.float32), jnp.zeros((r, DP - D - 1), jnp.float32)],
      axis=1)


def _proj_body(x_ref, pos_ref, wx_ref, wp_ref, b_ref, o_ref):
  acc = lax.dot_general(
      x_ref[...], wx_ref[...], (((1,), (0,)), ((), ())),
      precision=lax.Precision.HIGHEST, preferred_element_type=jnp.float32)
  acc += lax.dot_general(
      pos_ref[...], wp_ref[...], (((1,), (0,)), ((), ())),
      precision=lax.Precision.HIGHEST, preferred_element_type=jnp.float32)
  o_ref[...] = _pad_cols(acc + b_ref[...])


def _layer_body(h_ref, agg_ref, w_ref, b_ref, o_ref):
  agg = agg_ref[0] + agg_ref[1]
  deg = jnp.maximum(agg[:, D:D + 1], 1.0)
  m = h_ref[:, :D] + agg[:, :D] / deg
  out = lax.dot_general(
      m, w_ref[...], (((1,), (0,)), ((), ())),
      precision=lax.Precision.HIGHEST, preferred_element_type=jnp.float32)
  out = out + b_ref[...]
  o_ref[...] = _pad_cols(jnp.where(out >= 0.0, out, 0.01 * out))


_RB = 2000


def _proj(x, pos, wx, wp, b):
  return pl.pallas_call(
      _proj_body,
      grid=(N // _RB,),
      in_specs=[
          pl.BlockSpec((_RB, NODE_DIM), lambda i: (i, 0)),
          pl.BlockSpec((_RB, POS_DIM), lambda i: (i, 0)),
          pl.BlockSpec((NODE_DIM, D), lambda i: (0, 0)),
          pl.BlockSpec((POS_DIM, D), lambda i: (0, 0)),
          pl.BlockSpec((1, D), lambda i: (0, 0)),
      ],
      out_specs=pl.BlockSpec((_RB, DP), lambda i: (i, 0)),
      out_shape=jax.ShapeDtypeStruct((N, DP), jnp.float32),
  )(x, pos, wx, wp, b)


def _layer(h, agg, w, b):
  return pl.pallas_call(
      _layer_body,
      grid=(N // _RB,),
      in_specs=[
          pl.BlockSpec((_RB, DP), lambda i: (i, 0)),
          pl.BlockSpec((NUM_CORES, _RB, DP), lambda i: (0, i, 0)),
          pl.BlockSpec((D, D), lambda i: (0, 0)),
          pl.BlockSpec((1, D), lambda i: (0, 0)),
      ],
      out_specs=pl.BlockSpec((_RB, DP), lambda i: (i, 0)),
      out_shape=jax.ShapeDtypeStruct((N, DP), jnp.float32),
  )(h, agg, w, b)


def kernel(x, pos, edge_index, batch, Wp, bp, W0, b0, W1, b1, W2, b2, W3, b3):
  del batch, W3, b3
  src = edge_index[0]
  dst = edge_index[1]
  pad = EPAD - E
  src3 = jnp.concatenate(
      [src, jnp.zeros((pad,), jnp.int32)]).reshape(NTILES, NCH, CH)
  dst3 = jnp.concatenate(
      [dst, jnp.full((pad,), N, jnp.int32)]).reshape(NTILES, NCH, CH)
  z = jnp.zeros((NPAD, DP), jnp.float32)

  wx = Wp[POS_DIM:]
  wp = Wp[:POS_DIM]

  h0 = _proj(x, pos, wx, wp, bp.reshape(1, D))
  agg1 = _sc_agg(h0, src3, dst3, z)
  h1 = _layer(h0, agg1, W0, b0.reshape(1, D))
  agg2 = _sc_agg(h1, src3, dst3, z)
  h2 = _layer(h1, agg2, W1, b1.reshape(1, D))
  agg3 = _sc_agg(h2, src3, dst3, z)
  h3 = _layer(h2, agg3, W2, b2.reshape(1, D))
  return jnp.concatenate([h1[:, :D], h2[:, :D], h3[:, :D]], axis=-1)


# speedup vs baseline: 2.5318x; 2.5318x over previous
import jax
import jax.numpy as jnp
from jax import lax
from jax.experimental import pallas as pl
from jax.experimental.pallas import tpu as pltpu
from jax.experimental.pallas import tpu_sc as plsc

N = 10000
E = 320000
D = 64
DP = 128
NODE_DIM = 128
POS_DIM = 3

NUM_CORES = 2
NUM_SUBCORES = 16
NTILES = NUM_CORES * NUM_SUBCORES
CH = 125
NCH = 80
EPT = CH * NCH
NPAD = 10112
ZROWS = NPAD // NUM_SUBCORES
LAST_ROWS = N - 15 * ZROWS

_MESH = plsc.VectorSubcoreMesh(
    core_axis_name="c", subcore_axis_name="s",
    num_cores=NUM_CORES, num_subcores=NUM_SUBCORES)


def _sc_body(h_hbm, src_hbm, dst_hbm, z_hbm, agg_out,
             src_v, dst_v, rows0_v, agg_sh, sg0):
  c = lax.axis_index("c")
  s = lax.axis_index("s")
  wid = s * NUM_CORES + c

  zsl = pl.ds(s * ZROWS, ZROWS)
  pltpu.sync_copy(z_hbm.at[zsl], agg_sh.at[zsl])
  plsc.subcore_barrier()

  pltpu.sync_copy(src_hbm.at[wid], src_v)
  pltpu.sync_copy(dst_hbm.at[wid], dst_v)

  def chunk(i, carry):
    pltpu.async_copy(h_hbm.at[src_v.at[i]], rows0_v, sg0).wait()
    pltpu.sync_copy(rows0_v, agg_sh.at[dst_v.at[i]], add=True)
    return carry
  lax.fori_loop(0, NCH, chunk, 0)

  plsc.subcore_barrier()

  @pl.when(s < NUM_SUBCORES - 1)
  def _():
    osl = pl.ds(s * ZROWS, ZROWS)
    pltpu.sync_copy(agg_sh.at[osl], agg_out.at[c, osl])

  @pl.when(s == NUM_SUBCORES - 1)
  def _():
    osl = pl.ds(s * ZROWS, LAST_ROWS)
    pltpu.sync_copy(agg_sh.at[osl], agg_out.at[c, osl])


_sc_agg = pl.kernel(
    _sc_body,
    out_type=jax.ShapeDtypeStruct((NUM_CORES, N, DP), jnp.float32),
    mesh=_MESH,
    scratch_types=[
        pltpu.VMEM((NCH, CH), jnp.int32),
        pltpu.VMEM((NCH, CH), jnp.int32),
        pltpu.VMEM((CH, DP), jnp.float32),
        pltpu.VMEM_SHARED((NPAD, DP), jnp.float32),
        pltpu.SemaphoreType.DMA,
    ],
)


def _pad_cols(vals):
  r = vals.shape[0]
  return jnp.concatenate(
      [vals, jnp.ones((r, 1), jnp.float32), jnp.zeros((r, DP - D - 1), jnp.float32)],
      axis=1)


def _proj_body(x_ref, pos_ref, wx_ref, wp_ref, b_ref, o_ref):
  acc = lax.dot_general(
      x_ref[...], wx_ref[...], (((1,), (0,)), ((), ())),
      precision=lax.Precision.HIGHEST, preferred_element_type=jnp.float32)
  acc += lax.dot_general(
      pos_ref[...], wp_ref[...], (((1,), (0,)), ((), ())),
      precision=lax.Precision.HIGHEST, preferred_element_type=jnp.float32)
  o_ref[...] = _pad_cols(acc + b_ref[...])


def _layer_body(h_ref, agg_ref, w_ref, b_ref, o_ref):
  agg = agg_ref[0] + agg_ref[1]
  deg = jnp.maximum(agg[:, D:D + 1], 1.0)
  m = h_ref[:, :D] + agg[:, :D] / deg
  out = lax.dot_general(
      m, w_ref[...], (((1,), (0,)), ((), ())),
      precision=lax.Precision.HIGHEST, preferred_element_type=jnp.float32)
  out = out + b_ref[...]
  o_ref[...] = _pad_cols(jnp.where(out >= 0.0, out, 0.01 * out))


_RB = 2000


def _proj(x, pos, wx, wp, b):
  return pl.pallas_call(
      _proj_body,
      grid=(N // _RB,),
      in_specs=[
          pl.BlockSpec((_RB, NODE_DIM), lambda i: (i, 0)),
          pl.BlockSpec((_RB, POS_DIM), lambda i: (i, 0)),
          pl.BlockSpec((NODE_DIM, D), lambda i: (0, 0)),
          pl.BlockSpec((POS_DIM, D), lambda i: (0, 0)),
          pl.BlockSpec((1, D), lambda i: (0, 0)),
      ],
      out_specs=pl.BlockSpec((_RB, DP), lambda i: (i, 0)),
      out_shape=jax.ShapeDtypeStruct((N, DP), jnp.float32),
  )(x, pos, wx, wp, b)


def _layer(h, agg, w, b):
  return pl.pallas_call(
      _layer_body,
      grid=(N // _RB,),
      in_specs=[
          pl.BlockSpec((_RB, DP), lambda i: (i, 0)),
          pl.BlockSpec((NUM_CORES, _RB, DP), lambda i: (0, i, 0)),
          pl.BlockSpec((D, D), lambda i: (0, 0)),
          pl.BlockSpec((1, D), lambda i: (0, 0)),
      ],
      out_specs=pl.BlockSpec((_RB, DP), lambda i: (i, 0)),
      out_shape=jax.ShapeDtypeStruct((N, DP), jnp.float32),
  )(h, agg, w, b)


def kernel(x, pos, edge_index, batch, Wp, bp, W0, b0, W1, b1, W2, b2, W3, b3):
  del batch, W3, b3
  src3 = edge_index[0].reshape(NTILES, NCH, CH)
  dst3 = edge_index[1].reshape(NTILES, NCH, CH)
  z = jnp.zeros((NPAD, DP), jnp.float32)

  wx = Wp[POS_DIM:]
  wp = Wp[:POS_DIM]

  h0 = _proj(x, pos, wx, wp, bp.reshape(1, D))
  agg1 = _sc_agg(h0, src3, dst3, z)
  h1 = _layer(h0, agg1, W0, b0.reshape(1, D))
  agg2 = _sc_agg(h1, src3, dst3, z)
  h2 = _layer(h1, agg2, W1, b1.reshape(1, D))
  agg3 = _sc_agg(h2, src3, dst3, z)
  h3 = _layer(h2, agg3, W2, b2.reshape(1, D))
  return jnp.concatenate([h1[:, :D], h2[:, :D], h3[:, :D]], axis=-1)


# speedup vs baseline: 3.0718x; 1.2133x over previous
import jax
import jax.numpy as jnp
from jax import lax
from jax.experimental import pallas as pl
from jax.experimental.pallas import tpu as pltpu
from jax.experimental.pallas import tpu_sc as plsc

N = 10000
E = 320000
D = 64
DP = 128
NODE_DIM = 128
POS_DIM = 3

NUM_CORES = 2
NUM_SUBCORES = 16
NTILES = NUM_CORES * NUM_SUBCORES
CH = 125
NCH = 80
EPT = CH * NCH
GCH = 16
NG = NCH // GCH
NPAD = 10112
ZROWS = NPAD // NUM_SUBCORES
LAST_ROWS = N - 15 * ZROWS

_MESH = plsc.VectorSubcoreMesh(
    core_axis_name="c", subcore_axis_name="s",
    num_cores=NUM_CORES, num_subcores=NUM_SUBCORES)


def _sc_body(h_hbm, src_hbm, dst_hbm, z_hbm, agg_out,
             src_v, dst_v, rows0_v, rows1_v, agg_sh, sg0, sg1):
  c = lax.axis_index("c")
  s = lax.axis_index("s")
  wid = s * NUM_CORES + c

  zsl = pl.ds(s * ZROWS, ZROWS)
  pltpu.sync_copy(z_hbm.at[zsl], agg_sh.at[zsl])
  plsc.subcore_barrier()

  rows = (rows0_v, rows1_v)
  sg = (sg0, sg1)

  def start_gather(i, b):
    pltpu.async_copy(h_hbm.at[src_v.at[i]], rows[b], sg[b])

  def wait_gather(b):
    pltpu.make_async_copy(h_hbm.at[src_v.at[0]], rows[b], sg[b]).wait()

  def scatter(i, b):
    pltpu.sync_copy(rows[b], agg_sh.at[dst_v.at[i]], add=True)

  def group(g, carry):
    gsl = pl.ds(g * GCH, GCH)
    pltpu.sync_copy(src_hbm.at[wid, gsl], src_v)
    pltpu.sync_copy(dst_hbm.at[wid, gsl], dst_v)
    start_gather(0, 0)

    def pair(p, carry2):
      c0 = 2 * p
      wait_gather(0)
      start_gather(c0 + 1, 1)
      scatter(c0, 0)
      wait_gather(1)

      @pl.when(c0 + 2 < GCH)
      def _():
        start_gather(c0 + 2, 0)
      scatter(c0 + 1, 1)
      return carry2
    lax.fori_loop(0, GCH // 2, pair, 0)
    return carry
  lax.fori_loop(0, NG, group, 0)

  plsc.subcore_barrier()

  @pl.when(s < NUM_SUBCORES - 1)
  def _():
    osl = pl.ds(s * ZROWS, ZROWS)
    pltpu.sync_copy(agg_sh.at[osl], agg_out.at[c, osl])

  @pl.when(s == NUM_SUBCORES - 1)
  def _():
    osl = pl.ds(s * ZROWS, LAST_ROWS)
    pltpu.sync_copy(agg_sh.at[osl], agg_out.at[c, osl])


_sc_agg = pl.kernel(
    _sc_body,
    out_type=jax.ShapeDtypeStruct((NUM_CORES, N, DP), jnp.float32),
    mesh=_MESH,
    scratch_types=[
        pltpu.VMEM((GCH, CH), jnp.int32),
        pltpu.VMEM((GCH, CH), jnp.int32),
        pltpu.VMEM((CH, DP), jnp.float32),
        pltpu.VMEM((CH, DP), jnp.float32),
        pltpu.VMEM_SHARED((NPAD, DP), jnp.float32),
        pltpu.SemaphoreType.DMA,
        pltpu.SemaphoreType.DMA,
    ],
)


def _pad_cols(vals):
  r = vals.shape[0]
  return jnp.concatenate(
      [vals, jnp.ones((r, 1), jnp.float32), jnp.zeros((r, DP - D - 1), jnp.float32)],
      axis=1)


def _proj_body(x_ref, pos_ref, wx_ref, wp_ref, b_ref, o_ref):
  acc = lax.dot_general(
      x_ref[...], wx_ref[...], (((1,), (0,)), ((), ())),
      precision=lax.Precision.HIGHEST, preferred_element_type=jnp.float32)
  acc += lax.dot_general(
      pos_ref[...], wp_ref[...], (((1,), (0,)), ((), ())),
      precision=lax.Precision.HIGHEST, preferred_element_type=jnp.float32)
  o_ref[...] = _pad_cols(acc + b_ref[...])


def _layer_body(h_ref, agg_ref, w_ref, b_ref, o_ref):
  agg = agg_ref[0] + agg_ref[1]
  deg = jnp.maximum(agg[:, D:D + 1], 1.0)
  m = h_ref[:, :D] + agg[:, :D] / deg
  out = lax.dot_general(
      m, w_ref[...], (((1,), (0,)), ((), ())),
      precision=lax.Precision.HIGHEST, preferred_element_type=jnp.float32)
  out = out + b_ref[...]
  o_ref[...] = _pad_cols(jnp.where(out >= 0.0, out, 0.01 * out))


_RB = 2000


def _proj(x, pos, wx, wp, b):
  return pl.pallas_call(
      _proj_body,
      grid=(N // _RB,),
      in_specs=[
          pl.BlockSpec((_RB, NODE_DIM), lambda i: (i, 0)),
          pl.BlockSpec((_RB, POS_DIM), lambda i: (i, 0)),
          pl.BlockSpec((NODE_DIM, D), lambda i: (0, 0)),
          pl.BlockSpec((POS_DIM, D), lambda i: (0, 0)),
          pl.BlockSpec((1, D), lambda i: (0, 0)),
      ],
      out_specs=pl.BlockSpec((_RB, DP), lambda i: (i, 0)),
      out_shape=jax.ShapeDtypeStruct((N, DP), jnp.float32),
  )(x, pos, wx, wp, b)


def _layer(h, agg, w, b):
  return pl.pallas_call(
      _layer_body,
      grid=(N // _RB,),
      in_specs=[
          pl.BlockSpec((_RB, DP), lambda i: (i, 0)),
          pl.BlockSpec((NUM_CORES, _RB, DP), lambda i: (0, i, 0)),
          pl.BlockSpec((D, D), lambda i: (0, 0)),
          pl.BlockSpec((1, D), lambda i: (0, 0)),
      ],
      out_specs=pl.BlockSpec((_RB, DP), lambda i: (i, 0)),
      out_shape=jax.ShapeDtypeStruct((N, DP), jnp.float32),
  )(h, agg, w, b)


def kernel(x, pos, edge_index, batch, Wp, bp, W0, b0, W1, b1, W2, b2, W3, b3):
  del batch, W3, b3
  src3 = edge_index[0].reshape(NTILES, NCH, CH)
  dst3 = edge_index[1].reshape(NTILES, NCH, CH)
  z = jnp.zeros((NPAD, DP), jnp.float32)

  wx = Wp[POS_DIM:]
  wp = Wp[:POS_DIM]

  h0 = _proj(x, pos, wx, wp, bp.reshape(1, D))
  agg1 = _sc_agg(h0, src3, dst3, z)
  h1 = _layer(h0, agg1, W0, b0.reshape(1, D))
  agg2 = _sc_agg(h1, src3, dst3, z)
  h2 = _layer(h1, agg2, W1, b1.reshape(1, D))
  agg3 = _sc_agg(h2, src3, dst3, z)
  h3 = _layer(h2, agg3, W2, b2.reshape(1, D))
  return jnp.concatenate([h1[:, :D], h2[:, :D], h3[:, :D]], axis=-1)


# speedup vs baseline: 3.3102x; 1.0776x over previous
import jax
import jax.numpy as jnp
from jax import lax
from jax.experimental import pallas as pl
from jax.experimental.pallas import tpu as pltpu
from jax.experimental.pallas import tpu_sc as plsc

N = 10000
E = 320000
D = 64
DP = 128
NODE_DIM = 128
POS_DIM = 3

NUM_CORES = 2
NUM_SUBCORES = 16
NTILES = NUM_CORES * NUM_SUBCORES
CH = 125
NCH = 80
EPT = CH * NCH
GCH = 16
NG = NCH // GCH
NPAD = 10112
ZROWS = NPAD // NUM_SUBCORES
LAST_ROWS = N - 15 * ZROWS

_MESH = plsc.VectorSubcoreMesh(
    core_axis_name="c", subcore_axis_name="s",
    num_cores=NUM_CORES, num_subcores=NUM_SUBCORES)


def _sc_body(h_hbm, ei_hbm, z_hbm, agg_out,
             src_v, dst_v, rows0_v, rows1_v, agg_sh, sg0, sg1):
  c = lax.axis_index("c")
  s = lax.axis_index("s")
  wid = s * NUM_CORES + c

  zsl = pl.ds(s * ZROWS, ZROWS)
  pltpu.sync_copy(z_hbm.at[zsl], agg_sh.at[zsl])
  plsc.subcore_barrier()

  rows = (rows0_v, rows1_v)
  sg = (sg0, sg1)

  def start_gather(i, b):
    pltpu.async_copy(h_hbm.at[src_v.at[i]], rows[b], sg[b])

  def wait_gather(b):
    pltpu.make_async_copy(h_hbm.at[src_v.at[0]], rows[b], sg[b]).wait()

  def scatter(i, b):
    pltpu.sync_copy(rows[b], agg_sh.at[dst_v.at[i]], add=True)

  def group(g, carry):
    gsl = pl.ds(g * GCH, GCH)
    pltpu.sync_copy(ei_hbm.at[0, wid, gsl], src_v)
    pltpu.sync_copy(ei_hbm.at[1, wid, gsl], dst_v)
    start_gather(0, 0)

    def pair(p, carry2):
      c0 = 2 * p
      wait_gather(0)
      start_gather(c0 + 1, 1)
      scatter(c0, 0)
      wait_gather(1)

      @pl.when(c0 + 2 < GCH)
      def _():
        start_gather(c0 + 2, 0)
      scatter(c0 + 1, 1)
      return carry2
    lax.fori_loop(0, GCH // 2, pair, 0)
    return carry
  lax.fori_loop(0, NG, group, 0)

  plsc.subcore_barrier()

  @pl.when(s < NUM_SUBCORES - 1)
  def _():
    osl = pl.ds(s * ZROWS, ZROWS)
    pltpu.sync_copy(agg_sh.at[osl], agg_out.at[c, osl])

  @pl.when(s == NUM_SUBCORES - 1)
  def _():
    osl = pl.ds(s * ZROWS, LAST_ROWS)
    pltpu.sync_copy(agg_sh.at[osl], agg_out.at[c, osl])


_sc_agg = pl.kernel(
    _sc_body,
    out_type=jax.ShapeDtypeStruct((NUM_CORES, N, DP), jnp.float32),
    mesh=_MESH,
    scratch_types=[
        pltpu.VMEM((GCH, CH), jnp.int32),
        pltpu.VMEM((GCH, CH), jnp.int32),
        pltpu.VMEM((CH, DP), jnp.float32),
        pltpu.VMEM((CH, DP), jnp.float32),
        pltpu.VMEM_SHARED((NPAD, DP), jnp.float32),
        pltpu.SemaphoreType.DMA,
        pltpu.SemaphoreType.DMA,
    ],
)


def _pad_cols(vals):
  r = vals.shape[0]
  return jnp.concatenate(
      [vals, jnp.ones((r, 1), jnp.float32), jnp.zeros((r, DP - D - 1), jnp.float32)],
      axis=1)


def _proj_body(x_ref, pos_ref, wx_ref, wp_ref, b_ref, o_ref):
  acc = lax.dot_general(
      x_ref[...], wx_ref[...], (((1,), (0,)), ((), ())),
      precision=lax.Precision.HIGHEST, preferred_element_type=jnp.float32)
  acc += lax.dot_general(
      pos_ref[...], wp_ref[...], (((1,), (0,)), ((), ())),
      precision=lax.Precision.HIGHEST, preferred_element_type=jnp.float32)
  o_ref[...] = _pad_cols(acc + b_ref[...])


def _act(h_ref, agg_ref, w_ref, b_ref):
  agg = agg_ref[0] + agg_ref[1]
  deg = jnp.maximum(agg[:, D:D + 1], 1.0)
  m = h_ref[:, :D] + agg[:, :D] / deg
  out = lax.dot_general(
      m, w_ref[...], (((1,), (0,)), ((), ())),
      precision=lax.Precision.HIGHEST, preferred_element_type=jnp.float32)
  out = out + b_ref[...]
  return jnp.where(out >= 0.0, out, 0.01 * out)


_RB = 2000


def _proj(x, pos, wx, wp, b):
  return pl.pallas_call(
      _proj_body,
      grid=(N // _RB,),
      in_specs=[
          pl.BlockSpec((_RB, NODE_DIM), lambda i: (i, 0)),
          pl.BlockSpec((_RB, POS_DIM), lambda i: (i, 0)),
          pl.BlockSpec((NODE_DIM, D), lambda i: (0, 0)),
          pl.BlockSpec((POS_DIM, D), lambda i: (0, 0)),
          pl.BlockSpec((1, D), lambda i: (0, 0)),
      ],
      out_specs=pl.BlockSpec((_RB, DP), lambda i: (i, 0)),
      out_shape=jax.ShapeDtypeStruct((N, DP), jnp.float32),
  )(x, pos, wx, wp, b)


def _layer(l, h, agg, w, b, hcat=None):
  in_specs = [
      pl.BlockSpec((_RB, DP), lambda i: (i, 0)),
      pl.BlockSpec((NUM_CORES, _RB, DP), lambda i: (0, i, 0)),
      pl.BlockSpec((D, D), lambda i: (0, 0)),
      pl.BlockSpec((1, D), lambda i: (0, 0)),
  ]
  args = [h, agg, w, b]
  if l > 0:
    in_specs.append(pl.BlockSpec((_RB, 3 * D), lambda i: (i, 0)))
    args.append(hcat)
  last = l == 2

  def body(h_ref, agg_ref, w_ref, b_ref, *rest):
    act = _act(h_ref, agg_ref, w_ref, b_ref)
    if l == 0:
      o_ref, cat_ref = rest
      cat_ref[...] = jnp.concatenate(
          [act, jnp.zeros((act.shape[0], 2 * D), jnp.float32)], axis=1)
    elif l == 1:
      cat_in, o_ref, cat_ref = rest
      cat_ref[...] = jnp.concatenate(
          [cat_in[:, :D], act, jnp.zeros((act.shape[0], D), jnp.float32)],
          axis=1)
    else:
      cat_in, cat_ref = rest
      cat_ref[...] = jnp.concatenate([cat_in[:, :2 * D], act], axis=1)
      return
    o_ref[...] = _pad_cols(act)

  cat_spec = pl.BlockSpec((_RB, 3 * D), lambda i: (i, 0))
  cat_shape = jax.ShapeDtypeStruct((N, 3 * D), jnp.float32)
  if last:
    out_specs, out_shape = cat_spec, cat_shape
  else:
    out_specs = (pl.BlockSpec((_RB, DP), lambda i: (i, 0)), cat_spec)
    out_shape = (jax.ShapeDtypeStruct((N, DP), jnp.float32), cat_shape)

  return pl.pallas_call(
      body,
      grid=(N // _RB,),
      in_specs=in_specs,
      out_specs=out_specs,
      out_shape=out_shape,
  )(*args)


def kernel(x, pos, edge_index, batch, Wp, bp, W0, b0, W1, b1, W2, b2, W3, b3):
  del batch, W3, b3
  ei = edge_index.reshape(2, NTILES, NCH, CH)
  z = jnp.zeros((NPAD, DP), jnp.float32)

  wx = Wp[POS_DIM:]
  wp = Wp[:POS_DIM]

  h0 = _proj(x, pos, wx, wp, bp.reshape(1, D))
  agg1 = _sc_agg(h0, ei, z)
  h1, hcat = _layer(0, h0, agg1, W0, b0.reshape(1, D))
  agg2 = _sc_agg(h1, ei, z)
  h2, hcat = _layer(1, h1, agg2, W1, b1.reshape(1, D), hcat)
  agg3 = _sc_agg(h2, ei, z)
  hcat = _layer(2, h2, agg3, W2, b2.reshape(1, D), hcat)
  return hcat


# speedup vs baseline: 3.7433x; 1.1308x over previous
import jax
import jax.numpy as jnp
from jax import lax
from jax.experimental import pallas as pl
from jax.experimental.pallas import tpu as pltpu
from jax.experimental.pallas import tpu_sc as plsc

N = 10000
E = 320000
D = 64
NODE_DIM = 128
POS_DIM = 3

NUM_CORES = 2
NUM_SUBCORES = 16
NTILES = NUM_CORES * NUM_SUBCORES
CH = 125
NCH = 80
GCH = 16
NG = NCH // GCH
NPAD = 10112
ZROWS = NPAD // NUM_SUBCORES
LAST_ROWS = N - 15 * ZROWS
DW = 16

_MESH = plsc.VectorSubcoreMesh(
    core_axis_name="c", subcore_axis_name="s",
    num_cores=NUM_CORES, num_subcores=NUM_SUBCORES)
_SC_PARAMS = pltpu.CompilerParams(use_tc_tiling_on_sc=False)


def _row_split(s, agg_sh, agg_out, c):
  @pl.when(s < NUM_SUBCORES - 1)
  def _():
    osl = pl.ds(s * ZROWS, ZROWS)
    pltpu.sync_copy(agg_sh.at[osl], agg_out.at[c, osl])

  @pl.when(s == NUM_SUBCORES - 1)
  def _():
    osl = pl.ds(s * ZROWS, LAST_ROWS)
    pltpu.sync_copy(agg_sh.at[osl], agg_out.at[c, osl])


def _sc_body(h_hbm, ei_hbm, z_hbm, agg_out,
             src_v, dst_v, rows0_v, rows1_v, agg_sh, sg0, sg1):
  c = lax.axis_index("c")
  s = lax.axis_index("s")
  wid = s * NUM_CORES + c

  zsl = pl.ds(s * ZROWS, ZROWS)
  pltpu.sync_copy(z_hbm.at[zsl], agg_sh.at[zsl])
  plsc.subcore_barrier()

  rows = (rows0_v, rows1_v)
  sg = (sg0, sg1)

  def start_gather(i, b):
    pltpu.async_copy(h_hbm.at[src_v.at[i]], rows[b], sg[b])

  def wait_gather(b):
    pltpu.make_async_copy(h_hbm.at[src_v.at[0]], rows[b], sg[b]).wait()

  def scatter(i, b):
    pltpu.sync_copy(rows[b], agg_sh.at[dst_v.at[i]], add=True)

  def group(g, carry):
    gsl = pl.ds(g * GCH, GCH)
    pltpu.sync_copy(ei_hbm.at[0, wid, gsl], src_v)
    pltpu.sync_copy(ei_hbm.at[1, wid, gsl], dst_v)
    start_gather(0, 0)

    def pair(p, carry2):
      c0 = 2 * p
      wait_gather(0)
      start_gather(c0 + 1, 1)
      scatter(c0, 0)
      wait_gather(1)

      @pl.when(c0 + 2 < GCH)
      def _():
        start_gather(c0 + 2, 0)
      scatter(c0 + 1, 1)
      return carry2
    lax.fori_loop(0, GCH // 2, pair, 0)
    return carry
  lax.fori_loop(0, NG, group, 0)

  plsc.subcore_barrier()
  _row_split(s, agg_sh, agg_out, c)


_sc_agg = pl.kernel(
    _sc_body,
    out_type=jax.ShapeDtypeStruct((NUM_CORES, N, D), jnp.float32),
    mesh=_MESH,
    compiler_params=_SC_PARAMS,
    scratch_types=[
        pltpu.VMEM((GCH, CH), jnp.int32),
        pltpu.VMEM((GCH, CH), jnp.int32),
        pltpu.VMEM((CH, D), jnp.float32),
        pltpu.VMEM((CH, D), jnp.float32),
        pltpu.VMEM_SHARED((NPAD, D), jnp.float32),
        pltpu.SemaphoreType.DMA,
        pltpu.SemaphoreType.DMA,
    ],
)


def _deg_body(ei_hbm, zd_hbm, deg_out, dst_v, ones_v, deg_sh):
  c = lax.axis_index("c")
  s = lax.axis_index("s")
  wid = s * NUM_CORES + c

  zsl = pl.ds(s * ZROWS, ZROWS)
  pltpu.sync_copy(zd_hbm.at[zsl], deg_sh.at[zsl])

  def fill(i, carry):
    ones_v[i] = jnp.ones((DW,), jnp.float32)
    return carry
  lax.fori_loop(0, CH, fill, 0)
  plsc.subcore_barrier()

  def group(g, carry):
    gsl = pl.ds(g * GCH, GCH)
    pltpu.sync_copy(ei_hbm.at[1, wid, gsl], dst_v)

    def chunk(i, carry2):
      pltpu.sync_copy(ones_v, deg_sh.at[dst_v.at[i]], add=True)
      return carry2
    lax.fori_loop(0, GCH, chunk, 0)
    return carry
  lax.fori_loop(0, NG, group, 0)

  plsc.subcore_barrier()
  _row_split(s, deg_sh, deg_out, c)


_sc_deg = pl.kernel(
    _deg_body,
    out_type=jax.ShapeDtypeStruct((NUM_CORES, N, DW), jnp.float32),
    mesh=_MESH,
    compiler_params=_SC_PARAMS,
    scratch_types=[
        pltpu.VMEM((GCH, CH), jnp.int32),
        pltpu.VMEM((CH, DW), jnp.float32),
        pltpu.VMEM_SHARED((NPAD, DW), jnp.float32),
    ],
)


def _proj_body(x_ref, pos_ref, wx_ref, wp_ref, b_ref, o_ref):
  acc = lax.dot_general(
      x_ref[...], wx_ref[...], (((1,), (0,)), ((), ())),
      precision=lax.Precision.HIGHEST, preferred_element_type=jnp.float32)
  acc += lax.dot_general(
      pos_ref[...], wp_ref[...], (((1,), (0,)), ((), ())),
      precision=lax.Precision.HIGHEST, preferred_element_type=jnp.float32)
  o_ref[...] = acc + b_ref[...]


def _act(h_ref, agg_ref, deg_ref, w_ref, b_ref):
  agg = agg_ref[0] + agg_ref[1]
  deg = jnp.maximum(deg_ref[0, :, 0:1] + deg_ref[1, :, 0:1], 1.0)
  m = h_ref[...] + agg / deg
  out = lax.dot_general(
      m, w_ref[...], (((1,), (0,)), ((), ())),
      precision=lax.Precision.HIGHEST, preferred_element_type=jnp.float32)
  out = out + b_ref[...]
  return jnp.where(out >= 0.0, out, 0.01 * out)


_RB = 2000


def _proj(x, pos, wx, wp, b):
  return pl.pallas_call(
      _proj_body,
      grid=(N // _RB,),
      in_specs=[
          pl.BlockSpec((_RB, NODE_DIM), lambda i: (i, 0)),
          pl.BlockSpec((_RB, POS_DIM), lambda i: (i, 0)),
          pl.BlockSpec((NODE_DIM, D), lambda i: (0, 0)),
          pl.BlockSpec((POS_DIM, D), lambda i: (0, 0)),
          pl.BlockSpec((1, D), lambda i: (0, 0)),
      ],
      out_specs=pl.BlockSpec((_RB, D), lambda i: (i, 0)),
      out_shape=jax.ShapeDtypeStruct((N, D), jnp.float32),
  )(x, pos, wx, wp, b)


def _layer(l, h, agg, deg, w, b, hcat=None):
  in_specs = [
      pl.BlockSpec((_RB, D), lambda i: (i, 0)),
      pl.BlockSpec((NUM_CORES, _RB, D), lambda i: (0, i, 0)),
      pl.BlockSpec((NUM_CORES, _RB, DW), lambda i: (0, i, 0)),
      pl.BlockSpec((D, D), lambda i: (0, 0)),
      pl.BlockSpec((1, D), lambda i: (0, 0)),
  ]
  args = [h, agg, deg, w, b]
  if l > 0:
    in_specs.append(pl.BlockSpec((_RB, 3 * D), lambda i: (i, 0)))
    args.append(hcat)
  last = l == 2

  def body(h_ref, agg_ref, deg_ref, w_ref, b_ref, *rest):
    act = _act(h_ref, agg_ref, deg_ref, w_ref, b_ref)
    if l == 0:
      o_ref, cat_ref = rest
      cat_ref[...] = jnp.concatenate(
          [act, jnp.zeros((act.shape[0], 2 * D), jnp.float32)], axis=1)
    elif l == 1:
      cat_in, o_ref, cat_ref = rest
      cat_ref[...] = jnp.concatenate(
          [cat_in[:, :D], act, jnp.zeros((act.shape[0], D), jnp.float32)],
          axis=1)
    else:
      cat_in, cat_ref = rest
      cat_ref[...] = jnp.concatenate([cat_in[:, :2 * D], act], axis=1)
      return
    o_ref[...] = act

  cat_spec = pl.BlockSpec((_RB, 3 * D), lambda i: (i, 0))
  cat_shape = jax.ShapeDtypeStruct((N, 3 * D), jnp.float32)
  if last:
    out_specs, out_shape = cat_spec, cat_shape
  else:
    out_specs = (pl.BlockSpec((_RB, D), lambda i: (i, 0)), cat_spec)
    out_shape = (jax.ShapeDtypeStruct((N, D), jnp.float32), cat_shape)

  return pl.pallas_call(
      body,
      grid=(N // _RB,),
      in_specs=in_specs,
      out_specs=out_specs,
      out_shape=out_shape,
  )(*args)


def kernel(x, pos, edge_index, batch, Wp, bp, W0, b0, W1, b1, W2, b2, W3, b3):
  del batch, W3, b3
  ei = edge_index.reshape(2, NTILES, NCH, CH)
  z = jnp.zeros((NPAD, D), jnp.float32)
  zd = jnp.zeros((NPAD, DW), jnp.float32)

  wx = Wp[POS_DIM:]
  wp = Wp[:POS_DIM]

  deg = _sc_deg(ei, zd)
  h0 = _proj(x, pos, wx, wp, bp.reshape(1, D))
  agg1 = _sc_agg(h0, ei, z)
  h1, hcat = _layer(0, h0, agg1, deg, W0, b0.reshape(1, D))
  agg2 = _sc_agg(h1, ei, z)
  h2, hcat = _layer(1, h1, agg2, deg, W1, b1.reshape(1, D), hcat)
  agg3 = _sc_agg(h2, ei, z)
  hcat = _layer(2, h2, agg3, deg, W2, b2.reshape(1, D), hcat)
  return hcat


# speedup vs baseline: 4.7231x; 1.2617x over previous
import jax
import jax.numpy as jnp
from jax import lax
from jax.experimental import pallas as pl
from jax.experimental.pallas import tpu as pltpu
from jax.experimental.pallas import tpu_sc as plsc

N = 10000
E = 320000
D = 64
NODE_DIM = 128
POS_DIM = 3

NUM_CORES = 2
NUM_SUBCORES = 16
NTILES = NUM_CORES * NUM_SUBCORES
CH = 125
NCH = 80
GCH = 16
NG = NCH // GCH
NPAD = 10112
ZROWS = NPAD // NUM_SUBCORES
LAST_ROWS = N - 15 * ZROWS
DW = 16

_MESH = plsc.VectorSubcoreMesh(
    core_axis_name="c", subcore_axis_name="s",
    num_cores=NUM_CORES, num_subcores=NUM_SUBCORES)
_SC_PARAMS = pltpu.CompilerParams(use_tc_tiling_on_sc=False)


def _row_split(s, agg_sh, agg_out, c):
  @pl.when(s < NUM_SUBCORES - 1)
  def _():
    osl = pl.ds(s * ZROWS, ZROWS)
    pltpu.sync_copy(agg_sh.at[osl], agg_out.at[c, osl])

  @pl.when(s == NUM_SUBCORES - 1)
  def _():
    osl = pl.ds(s * ZROWS, LAST_ROWS)
    pltpu.sync_copy(agg_sh.at[osl], agg_out.at[c, osl])


def _sc_body(h_hbm, ei_hbm, z_hbm, agg_out,
             src_v, dst_v, rows0_v, rows1_v, rows2_v, rows3_v, agg_sh,
             sg0, sg1, sg2, sg3, ss0, ss1, ss2, ss3):
  c = lax.axis_index("c")
  s = lax.axis_index("s")
  wid = s * NUM_CORES + c

  zsl = pl.ds(s * ZROWS, ZROWS)
  pltpu.sync_copy(z_hbm.at[zsl], agg_sh.at[zsl])
  plsc.subcore_barrier()

  pltpu.sync_copy(ei_hbm.at[0, wid], src_v)
  pltpu.sync_copy(ei_hbm.at[1, wid], dst_v)

  rows = (rows0_v, rows1_v, rows2_v, rows3_v)
  sg = (sg0, sg1, sg2, sg3)
  ss = (ss0, ss1, ss2, ss3)

  def g(i, b):
    pltpu.async_copy(h_hbm.at[src_v.at[i]], rows[b], sg[b])

  def wg(b):
    pltpu.make_async_copy(h_hbm.at[src_v.at[0]], rows[b], sg[b]).wait()

  def sc(i, b):
    pltpu.async_copy(rows[b], agg_sh.at[dst_v.at[i]], ss[b], add=True)

  def ws(b):
    pltpu.make_async_copy(rows[b], agg_sh.at[dst_v.at[0]], ss[b]).wait()

  g(0, 0)
  g(1, 1)
  wg(0); sc(0, 0); g(2, 2)
  wg(1); sc(1, 1); g(3, 3)
  wg(2); sc(2, 2); ws(0); g(4, 0)
  wg(3); sc(3, 3); ws(1); g(5, 1)

  def quad(q, carry):
    c0 = 4 * q
    wg(0); sc(c0, 0); ws(2); g(c0 + 2, 2)
    wg(1); sc(c0 + 1, 1); ws(3); g(c0 + 3, 3)
    wg(2); sc(c0 + 2, 2); ws(0); g(c0 + 4, 0)
    wg(3); sc(c0 + 3, 3); ws(1); g(c0 + 5, 1)
    return carry
  lax.fori_loop(1, NCH // 4 - 1, quad, 0)

  c0 = NCH - 4
  wg(0); sc(c0, 0); ws(2); g(c0 + 2, 2)
  wg(1); sc(c0 + 1, 1); ws(3); g(c0 + 3, 3)
  wg(2); sc(c0 + 2, 2)
  wg(3); sc(c0 + 3, 3)
  ws(0); ws(1); ws(2); ws(3)

  plsc.subcore_barrier()
  _row_split(s, agg_sh, agg_out, c)


_sc_agg = pl.kernel(
    _sc_body,
    out_type=jax.ShapeDtypeStruct((NUM_CORES, N, D), jnp.float32),
    mesh=_MESH,
    compiler_params=_SC_PARAMS,
    scratch_types=(
        [pltpu.VMEM((NCH, CH), jnp.int32)] * 2
        + [pltpu.VMEM((CH, D), jnp.float32)] * 4
        + [pltpu.VMEM_SHARED((NPAD, D), jnp.float32)]
        + [pltpu.SemaphoreType.DMA] * 8
    ),
)


def _deg_body(ei_hbm, zd_hbm, deg_out, dst_v, ones_v, deg_sh):
  c = lax.axis_index("c")
  s = lax.axis_index("s")
  wid = s * NUM_CORES + c

  zsl = pl.ds(s * ZROWS, ZROWS)
  pltpu.sync_copy(zd_hbm.at[zsl], deg_sh.at[zsl])

  def fill(i, carry):
    ones_v[i] = jnp.ones((DW,), jnp.float32)
    return carry
  lax.fori_loop(0, CH, fill, 0)
  plsc.subcore_barrier()

  def group(g, carry):
    gsl = pl.ds(g * GCH, GCH)
    pltpu.sync_copy(ei_hbm.at[1, wid, gsl], dst_v)

    def chunk(i, carry2):
      pltpu.sync_copy(ones_v, deg_sh.at[dst_v.at[i]], add=True)
      return carry2
    lax.fori_loop(0, GCH, chunk, 0)
    return carry
  lax.fori_loop(0, NG, group, 0)

  plsc.subcore_barrier()
  _row_split(s, deg_sh, deg_out, c)


_sc_deg = pl.kernel(
    _deg_body,
    out_type=jax.ShapeDtypeStruct((NUM_CORES, N, DW), jnp.float32),
    mesh=_MESH,
    compiler_params=_SC_PARAMS,
    scratch_types=[
        pltpu.VMEM((GCH, CH), jnp.int32),
        pltpu.VMEM((CH, DW), jnp.float32),
        pltpu.VMEM_SHARED((NPAD, DW), jnp.float32),
    ],
)


def _proj_body(x_ref, pos_ref, wx_ref, wp_ref, b_ref, o_ref):
  acc = lax.dot_general(
      x_ref[...], wx_ref[...], (((1,), (0,)), ((), ())),
      precision=lax.Precision.HIGHEST, preferred_element_type=jnp.float32)
  acc += lax.dot_general(
      pos_ref[...], wp_ref[...], (((1,), (0,)), ((), ())),
      precision=lax.Precision.HIGHEST, preferred_element_type=jnp.float32)
  o_ref[...] = acc + b_ref[...]


def _act(h_ref, agg_ref, deg_ref, w_ref, b_ref):
  agg = agg_ref[0] + agg_ref[1]
  deg = jnp.maximum(deg_ref[0, :, 0:1] + deg_ref[1, :, 0:1], 1.0)
  m = h_ref[...] + agg / deg
  out = lax.dot_general(
      m, w_ref[...], (((1,), (0,)), ((), ())),
      precision=lax.Precision.HIGHEST, preferred_element_type=jnp.float32)
  out = out + b_ref[...]
  return jnp.where(out >= 0.0, out, 0.01 * out)


_RB = 2000


def _proj(x, pos, wx, wp, b):
  return pl.pallas_call(
      _proj_body,
      grid=(N // _RB,),
      in_specs=[
          pl.BlockSpec((_RB, NODE_DIM), lambda i: (i, 0)),
          pl.BlockSpec((_RB, POS_DIM), lambda i: (i, 0)),
          pl.BlockSpec((NODE_DIM, D), lambda i: (0, 0)),
          pl.BlockSpec((POS_DIM, D), lambda i: (0, 0)),
          pl.BlockSpec((1, D), lambda i: (0, 0)),
      ],
      out_specs=pl.BlockSpec((_RB, D), lambda i: (i, 0)),
      out_shape=jax.ShapeDtypeStruct((N, D), jnp.float32),
  )(x, pos, wx, wp, b)


def _layer(l, h, agg, deg, w, b, hcat=None):
  in_specs = [
      pl.BlockSpec((_RB, D), lambda i: (i, 0)),
      pl.BlockSpec((NUM_CORES, _RB, D), lambda i: (0, i, 0)),
      pl.BlockSpec((NUM_CORES, _RB, DW), lambda i: (0, i, 0)),
      pl.BlockSpec((D, D), lambda i: (0, 0)),
      pl.BlockSpec((1, D), lambda i: (0, 0)),
  ]
  args = [h, agg, deg, w, b]
  if l > 0:
    in_specs.append(pl.BlockSpec((_RB, 3 * D), lambda i: (i, 0)))
    args.append(hcat)
  last = l == 2

  def body(h_ref, agg_ref, deg_ref, w_ref, b_ref, *rest):
    act = _act(h_ref, agg_ref, deg_ref, w_ref, b_ref)
    if l == 0:
      o_ref, cat_ref = rest
      cat_ref[...] = jnp.concatenate(
          [act, jnp.zeros((act.shape[0], 2 * D), jnp.float32)], axis=1)
    elif l == 1:
      cat_in, o_ref, cat_ref = rest
      cat_ref[...] = jnp.concatenate(
          [cat_in[:, :D], act, jnp.zeros((act.shape[0], D), jnp.float32)],
          axis=1)
    else:
      cat_in, cat_ref = rest
      cat_ref[...] = jnp.concatenate([cat_in[:, :2 * D], act], axis=1)
      return
    o_ref[...] = act

  cat_spec = pl.BlockSpec((_RB, 3 * D), lambda i: (i, 0))
  cat_shape = jax.ShapeDtypeStruct((N, 3 * D), jnp.float32)
  if last:
    out_specs, out_shape = cat_spec, cat_shape
  else:
    out_specs = (pl.BlockSpec((_RB, D), lambda i: (i, 0)), cat_spec)
    out_shape = (jax.ShapeDtypeStruct((N, D), jnp.float32), cat_shape)

  return pl.pallas_call(
      body,
      grid=(N // _RB,),
      in_specs=in_specs,
      out_specs=out_specs,
      out_shape=out_shape,
  )(*args)


def kernel(x, pos, edge_index, batch, Wp, bp, W0, b0, W1, b1, W2, b2, W3, b3):
  del batch, W3, b3
  ei = edge_index.reshape(2, NTILES, NCH, CH)
  z = jnp.zeros((NPAD, D), jnp.float32)
  zd = jnp.zeros((NPAD, DW), jnp.float32)

  wx = Wp[POS_DIM:]
  wp = Wp[:POS_DIM]

  deg = _sc_deg(ei, zd)
  h0 = _proj(x, pos, wx, wp, bp.reshape(1, D))
  agg1 = _sc_agg(h0, ei, z)
  h1, hcat = _layer(0, h0, agg1, deg, W0, b0.reshape(1, D))
  agg2 = _sc_agg(h1, ei, z)
  h2, hcat = _layer(1, h1, agg2, deg, W1, b1.reshape(1, D), hcat)
  agg3 = _sc_agg(h2, ei, z)
  hcat = _layer(2, h2, agg3, deg, W2, b2.reshape(1, D), hcat)
  return hcat


# speedup vs baseline: 5.0162x; 1.0621x over previous
import jax
import jax.numpy as jnp
from jax import lax
from jax.experimental import pallas as pl
from jax.experimental.pallas import tpu as pltpu
from jax.experimental.pallas import tpu_sc as plsc

N = 10000
E = 320000
D = 64
NODE_DIM = 128
POS_DIM = 3

NUM_CORES = 2
NUM_SUBCORES = 16
NTILES = NUM_CORES * NUM_SUBCORES
CH = 125
NCH = 80
GCH = 16
NG = NCH // GCH
NPAD = 10112
ZROWS = NPAD // NUM_SUBCORES
LAST_ROWS = N - 15 * ZROWS
DW = 16

_MESH = plsc.VectorSubcoreMesh(
    core_axis_name="c", subcore_axis_name="s",
    num_cores=NUM_CORES, num_subcores=NUM_SUBCORES)
_SC_PARAMS = pltpu.CompilerParams(use_tc_tiling_on_sc=False)


def _row_split(s, agg_sh, agg_out, c):
  @pl.when(s < NUM_SUBCORES - 1)
  def _():
    osl = pl.ds(s * ZROWS, ZROWS)
    pltpu.sync_copy(agg_sh.at[osl], agg_out.at[c, osl])

  @pl.when(s == NUM_SUBCORES - 1)
  def _():
    osl = pl.ds(s * ZROWS, LAST_ROWS)
    pltpu.sync_copy(agg_sh.at[osl], agg_out.at[c, osl])


NB = 8
PF = 4


def _sc_body(h_hbm, ei_hbm, z_hbm, agg_out, src_v, dst_v, *rest):
  rows = rest[:NB]
  agg_sh = rest[NB]
  sg = rest[NB + 1:2 * NB + 1]
  ss = rest[2 * NB + 1:]
  c = lax.axis_index("c")
  s = lax.axis_index("s")
  wid = s * NUM_CORES + c

  zsl = pl.ds(s * ZROWS, ZROWS)
  pltpu.sync_copy(z_hbm.at[zsl], agg_sh.at[zsl])
  plsc.subcore_barrier()

  pltpu.sync_copy(ei_hbm.at[0, wid], src_v)
  pltpu.sync_copy(ei_hbm.at[1, wid], dst_v)

  def g(i, b):
    pltpu.async_copy(h_hbm.at[src_v.at[i]], rows[b], sg[b])

  def wg(b):
    pltpu.make_async_copy(h_hbm.at[src_v.at[0]], rows[b], sg[b]).wait()

  def sc(i, b):
    pltpu.async_copy(rows[b], agg_sh.at[dst_v.at[i]], ss[b], add=True)

  def ws(b):
    pltpu.make_async_copy(rows[b], agg_sh.at[dst_v.at[0]], ss[b]).wait()

  for i in range(PF):
    g(i, i)
  for c0 in range(PF):
    wg(c0 % NB); sc(c0, c0 % NB); g(c0 + PF, (c0 + PF) % NB)
  for c0 in range(PF, NB):
    wg(c0 % NB); sc(c0, c0 % NB); ws((c0 + PF) % NB); g(c0 + PF, (c0 + PF) % NB)

  def ring(q, carry):
    c0 = NB * q
    for r in range(NB):
      b = r
      tb = (r + PF) % NB
      wg(b); sc(c0 + r, b); ws(tb)
      g(c0 + r + PF, tb)
    return carry
  lax.fori_loop(1, NCH // NB - 1, ring, 0)

  for r in range(NB):
    c0 = NCH - NB + r
    b = c0 % NB
    wg(b); sc(c0, b)
    if c0 + PF < NCH:
      ws((c0 + PF) % NB); g(c0 + PF, (c0 + PF) % NB)
  for b in range(NB):
    ws(b)

  plsc.subcore_barrier()
  _row_split(s, agg_sh, agg_out, c)


_sc_agg = pl.kernel(
    _sc_body,
    out_type=jax.ShapeDtypeStruct((NUM_CORES, N, D), jnp.float32),
    mesh=_MESH,
    compiler_params=_SC_PARAMS,
    scratch_types=(
        [pltpu.VMEM((NCH, CH), jnp.int32)] * 2
        + [pltpu.VMEM((CH, D), jnp.float32)] * NB
        + [pltpu.VMEM_SHARED((NPAD, D), jnp.float32)]
        + [pltpu.SemaphoreType.DMA] * (2 * NB)
    ),
)


def _deg_body(ei_hbm, zd_hbm, deg_out, dst_v, ones_v, deg_sh):
  c = lax.axis_index("c")
  s = lax.axis_index("s")
  wid = s * NUM_CORES + c

  zsl = pl.ds(s * ZROWS, ZROWS)
  pltpu.sync_copy(zd_hbm.at[zsl], deg_sh.at[zsl])

  def fill(i, carry):
    ones_v[i] = jnp.ones((DW,), jnp.float32)
    return carry
  lax.fori_loop(0, CH, fill, 0)
  plsc.subcore_barrier()

  def group(g, carry):
    gsl = pl.ds(g * GCH, GCH)
    pltpu.sync_copy(ei_hbm.at[1, wid, gsl], dst_v)

    def chunk(i, carry2):
      pltpu.sync_copy(ones_v, deg_sh.at[dst_v.at[i]], add=True)
      return carry2
    lax.fori_loop(0, GCH, chunk, 0)
    return carry
  lax.fori_loop(0, NG, group, 0)

  plsc.subcore_barrier()
  _row_split(s, deg_sh, deg_out, c)


_sc_deg = pl.kernel(
    _deg_body,
    out_type=jax.ShapeDtypeStruct((NUM_CORES, N, DW), jnp.float32),
    mesh=_MESH,
    compiler_params=_SC_PARAMS,
    scratch_types=[
        pltpu.VMEM((GCH, CH), jnp.int32),
        pltpu.VMEM((CH, DW), jnp.float32),
        pltpu.VMEM_SHARED((NPAD, DW), jnp.float32),
    ],
)


def _proj_body(x_ref, pos_ref, wx_ref, wp_ref, b_ref, o_ref):
  acc = lax.dot_general(
      x_ref[...], wx_ref[...], (((1,), (0,)), ((), ())),
      precision=lax.Precision.HIGHEST, preferred_element_type=jnp.float32)
  acc += lax.dot_general(
      pos_ref[...], wp_ref[...], (((1,), (0,)), ((), ())),
      precision=lax.Precision.HIGHEST, preferred_element_type=jnp.float32)
  o_ref[...] = acc + b_ref[...]


def _act(h_ref, agg_ref, deg_ref, w_ref, b_ref):
  agg = agg_ref[0] + agg_ref[1]
  deg = jnp.maximum(deg_ref[0, :, 0:1] + deg_ref[1, :, 0:1], 1.0)
  m = h_ref[...] + agg / deg
  out = lax.dot_general(
      m, w_ref[...], (((1,), (0,)), ((), ())),
      precision=lax.Precision.HIGHEST, preferred_element_type=jnp.float32)
  out = out + b_ref[...]
  return jnp.where(out >= 0.0, out, 0.01 * out)


_RB = 2000


def _proj(x, pos, wx, wp, b):
  return pl.pallas_call(
      _proj_body,
      grid=(N // _RB,),
      in_specs=[
          pl.BlockSpec((_RB, NODE_DIM), lambda i: (i, 0)),
          pl.BlockSpec((_RB, POS_DIM), lambda i: (i, 0)),
          pl.BlockSpec((NODE_DIM, D), lambda i: (0, 0)),
          pl.BlockSpec((POS_DIM, D), lambda i: (0, 0)),
          pl.BlockSpec((1, D), lambda i: (0, 0)),
      ],
      out_specs=pl.BlockSpec((_RB, D), lambda i: (i, 0)),
      out_shape=jax.ShapeDtypeStruct((N, D), jnp.float32),
  )(x, pos, wx, wp, b)


def _layer(l, h, agg, deg, w, b, hcat=None):
  in_specs = [
      pl.BlockSpec((_RB, D), lambda i: (i, 0)),
      pl.BlockSpec((NUM_CORES, _RB, D), lambda i: (0, i, 0)),
      pl.BlockSpec((NUM_CORES, _RB, DW), lambda i: (0, i, 0)),
      pl.BlockSpec((D, D), lambda i: (0, 0)),
      pl.BlockSpec((1, D), lambda i: (0, 0)),
  ]
  args = [h, agg, deg, w, b]
  if l > 0:
    in_specs.append(pl.BlockSpec((_RB, 3 * D), lambda i: (i, 0)))
    args.append(hcat)
  last = l == 2

  def body(h_ref, agg_ref, deg_ref, w_ref, b_ref, *rest):
    act = _act(h_ref, agg_ref, deg_ref, w_ref, b_ref)
    if l == 0:
      o_ref, cat_ref = rest
      cat_ref[...] = jnp.concatenate(
          [act, jnp.zeros((act.shape[0], 2 * D), jnp.float32)], axis=1)
    elif l == 1:
      cat_in, o_ref, cat_ref = rest
      cat_ref[...] = jnp.concatenate(
          [cat_in[:, :D], act, jnp.zeros((act.shape[0], D), jnp.float32)],
          axis=1)
    else:
      cat_in, cat_ref = rest
      cat_ref[...] = jnp.concatenate([cat_in[:, :2 * D], act], axis=1)
      return
    o_ref[...] = act

  cat_spec = pl.BlockSpec((_RB, 3 * D), lambda i: (i, 0))
  cat_shape = jax.ShapeDtypeStruct((N, 3 * D), jnp.float32)
  if last:
    out_specs, out_shape = cat_spec, cat_shape
  else:
    out_specs = (pl.BlockSpec((_RB, D), lambda i: (i, 0)), cat_spec)
    out_shape = (jax.ShapeDtypeStruct((N, D), jnp.float32), cat_shape)

  return pl.pallas_call(
      body,
      grid=(N // _RB,),
      in_specs=in_specs,
      out_specs=out_specs,
      out_shape=out_shape,
  )(*args)


def kernel(x, pos, edge_index, batch, Wp, bp, W0, b0, W1, b1, W2, b2, W3, b3):
  del batch, W3, b3
  ei = edge_index.reshape(2, NTILES, NCH, CH)
  z = jnp.zeros((NPAD, D), jnp.float32)
  zd = jnp.zeros((NPAD, DW), jnp.float32)

  wx = Wp[POS_DIM:]
  wp = Wp[:POS_DIM]

  deg = _sc_deg(ei, zd)
  h0 = _proj(x, pos, wx, wp, bp.reshape(1, D))
  agg1 = _sc_agg(h0, ei, z)
  h1, hcat = _layer(0, h0, agg1, deg, W0, b0.reshape(1, D))
  agg2 = _sc_agg(h1, ei, z)
  h2, hcat = _layer(1, h1, agg2, deg, W1, b1.reshape(1, D), hcat)
  agg3 = _sc_agg(h2, ei, z)
  hcat = _layer(2, h2, agg3, deg, W2, b2.reshape(1, D), hcat)
  return hcat


# speedup vs baseline: 5.1375x; 1.0242x over previous
import jax
import jax.numpy as jnp
from jax import lax
from jax.experimental import pallas as pl
from jax.experimental.pallas import tpu as pltpu
from jax.experimental.pallas import tpu_sc as plsc

N = 10000
E = 320000
D = 64
NODE_DIM = 128
POS_DIM = 3

NUM_CORES = 2
NUM_SUBCORES = 16
NTILES = NUM_CORES * NUM_SUBCORES
CH = 125
NCH = 80
GCH = 16
NG = NCH // GCH
NPAD = 10112
ZROWS = NPAD // NUM_SUBCORES
LAST_ROWS = N - 15 * ZROWS
DW = 16

_MESH = plsc.VectorSubcoreMesh(
    core_axis_name="c", subcore_axis_name="s",
    num_cores=NUM_CORES, num_subcores=NUM_SUBCORES)
_SC_PARAMS = pltpu.CompilerParams(use_tc_tiling_on_sc=False)


def _row_split(s, agg_sh, agg_out, c):
  @pl.when(s < NUM_SUBCORES - 1)
  def _():
    osl = pl.ds(s * ZROWS, ZROWS)
    pltpu.sync_copy(agg_sh.at[osl], agg_out.at[c, osl])

  @pl.when(s == NUM_SUBCORES - 1)
  def _():
    osl = pl.ds(s * ZROWS, LAST_ROWS)
    pltpu.sync_copy(agg_sh.at[osl], agg_out.at[c, osl])


NB = 8


def _sc_body(nb, with_deg, *refs):
  if with_deg:
    (h_hbm, ei_hbm, z_hbm, zd_hbm, agg_out, deg_out, src_v, dst_v), rest = (
        refs[:8], refs[8:])
  else:
    (h_hbm, ei_hbm, z_hbm, agg_out, src_v, dst_v), rest = refs[:6], refs[6:]
  rows = rest[:nb]
  rest = rest[nb:]
  if with_deg:
    ones_v, deg_sh = rest[0], rest[2]
    agg_sh = rest[1]
    rest = rest[3:]
  else:
    agg_sh = rest[0]
    rest = rest[1:]
  sg = rest[:nb]
  ss = rest[nb:]
  pf = nb // 2
  c = lax.axis_index("c")
  s = lax.axis_index("s")
  wid = s * NUM_CORES + c

  zsl = pl.ds(s * ZROWS, ZROWS)
  pltpu.sync_copy(z_hbm.at[zsl], agg_sh.at[zsl])
  if with_deg:
    pltpu.sync_copy(zd_hbm.at[zsl], deg_sh.at[zsl])

    def fill(i, carry):
      ones_v[i] = jnp.ones((DW,), jnp.float32)
      return carry
    lax.fori_loop(0, CH, fill, 0)
  plsc.subcore_barrier()

  pltpu.sync_copy(ei_hbm.at[0, wid], src_v)
  pltpu.sync_copy(ei_hbm.at[1, wid], dst_v)

  def g(i, b):
    pltpu.async_copy(h_hbm.at[src_v.at[i]], rows[b], sg[b])

  def wg(b):
    pltpu.make_async_copy(h_hbm.at[src_v.at[0]], rows[b], sg[b]).wait()

  def sc(i, b):
    pltpu.async_copy(rows[b], agg_sh.at[dst_v.at[i]], ss[b], add=True)
    if with_deg:
      pltpu.sync_copy(ones_v, deg_sh.at[dst_v.at[i]], add=True)

  def ws(b):
    pltpu.make_async_copy(rows[b], agg_sh.at[dst_v.at[0]], ss[b]).wait()

  for i in range(pf):
    g(i, i)
  for c0 in range(pf):
    wg(c0 % nb); sc(c0, c0 % nb); g(c0 + pf, (c0 + pf) % nb)
  for c0 in range(pf, nb):
    wg(c0 % nb); sc(c0, c0 % nb); ws((c0 + pf) % nb); g(c0 + pf, (c0 + pf) % nb)

  def ring(q, carry):
    c0 = nb * q
    for r in range(nb):
      tb = (r + pf) % nb
      wg(r); sc(c0 + r, r); ws(tb)
      g(c0 + r + pf, tb)
    return carry
  lax.fori_loop(1, NCH // nb - 1, ring, 0)

  for r in range(NCH % nb + nb):
    c0 = (NCH // nb - 1) * nb + r
    b = c0 % nb
    wg(b); sc(c0, b)
    if c0 + pf < NCH:
      ws((c0 + pf) % nb); g(c0 + pf, (c0 + pf) % nb)
  for b in range(nb):
    ws(b)

  plsc.subcore_barrier()
  _row_split(s, agg_sh, agg_out, c)
  if with_deg:
    _row_split(s, deg_sh, deg_out, c)


def _make_sc(nb, with_deg):
  import functools
  out_type = [jax.ShapeDtypeStruct((NUM_CORES, N, D), jnp.float32)]
  scratch = [pltpu.VMEM((NCH, CH), jnp.int32)] * 2
  scratch += [pltpu.VMEM((CH, D), jnp.float32)] * nb
  if with_deg:
    out_type.append(jax.ShapeDtypeStruct((NUM_CORES, N, DW), jnp.float32))
    scratch.append(pltpu.VMEM((CH, DW), jnp.float32))
  scratch.append(pltpu.VMEM_SHARED((NPAD, D), jnp.float32))
  if with_deg:
    scratch.append(pltpu.VMEM_SHARED((NPAD, DW), jnp.float32))
  scratch += [pltpu.SemaphoreType.DMA] * (2 * nb)
  return pl.kernel(
      functools.partial(_sc_body, nb, with_deg),
      out_type=tuple(out_type) if with_deg else out_type[0],
      mesh=_MESH,
      compiler_params=_SC_PARAMS,
      scratch_types=scratch,
  )


_sc_agg = _make_sc(NB, False)
_sc_agg_deg = _make_sc(6, True)


def _proj_body(x_ref, pos_ref, wx_ref, wp_ref, b_ref, o_ref):
  acc = lax.dot_general(
      x_ref[...], wx_ref[...], (((1,), (0,)), ((), ())),
      precision=lax.Precision.HIGHEST, preferred_element_type=jnp.float32)
  acc += lax.dot_general(
      pos_ref[...], wp_ref[...], (((1,), (0,)), ((), ())),
      precision=lax.Precision.HIGHEST, preferred_element_type=jnp.float32)
  o_ref[...] = acc + b_ref[...]


def _act(h_ref, agg_ref, deg_ref, w_ref, b_ref):
  agg = agg_ref[0] + agg_ref[1]
  deg = jnp.maximum(deg_ref[0, :, 0:1] + deg_ref[1, :, 0:1], 1.0)
  m = h_ref[...] + agg / deg
  out = lax.dot_general(
      m, w_ref[...], (((1,), (0,)), ((), ())),
      precision=lax.Precision.HIGHEST, preferred_element_type=jnp.float32)
  out = out + b_ref[...]
  return jnp.where(out >= 0.0, out, 0.01 * out)


_RB = 2000


def _proj(x, pos, wx, wp, b):
  return pl.pallas_call(
      _proj_body,
      grid=(N // _RB,),
      in_specs=[
          pl.BlockSpec((_RB, NODE_DIM), lambda i: (i, 0)),
          pl.BlockSpec((_RB, POS_DIM), lambda i: (i, 0)),
          pl.BlockSpec((NODE_DIM, D), lambda i: (0, 0)),
          pl.BlockSpec((POS_DIM, D), lambda i: (0, 0)),
          pl.BlockSpec((1, D), lambda i: (0, 0)),
      ],
      out_specs=pl.BlockSpec((_RB, D), lambda i: (i, 0)),
      out_shape=jax.ShapeDtypeStruct((N, D), jnp.float32),
  )(x, pos, wx, wp, b)


def _layer(l, h, agg, deg, w, b, hcat=None):
  in_specs = [
      pl.BlockSpec((_RB, D), lambda i: (i, 0)),
      pl.BlockSpec((NUM_CORES, _RB, D), lambda i: (0, i, 0)),
      pl.BlockSpec((NUM_CORES, _RB, DW), lambda i: (0, i, 0)),
      pl.BlockSpec((D, D), lambda i: (0, 0)),
      pl.BlockSpec((1, D), lambda i: (0, 0)),
  ]
  args = [h, agg, deg, w, b]
  if l > 0:
    in_specs.append(pl.BlockSpec((_RB, 3 * D), lambda i: (i, 0)))
    args.append(hcat)
  last = l == 2

  def body(h_ref, agg_ref, deg_ref, w_ref, b_ref, *rest):
    act = _act(h_ref, agg_ref, deg_ref, w_ref, b_ref)
    if l == 0:
      o_ref, cat_ref = rest
      cat_ref[...] = jnp.concatenate(
          [act, jnp.zeros((act.shape[0], 2 * D), jnp.float32)], axis=1)
    elif l == 1:
      cat_in, o_ref, cat_ref = rest
      cat_ref[...] = jnp.concatenate(
          [cat_in[:, :D], act, jnp.zeros((act.shape[0], D), jnp.float32)],
          axis=1)
    else:
      cat_in, cat_ref = rest
      cat_ref[...] = jnp.concatenate([cat_in[:, :2 * D], act], axis=1)
      return
    o_ref[...] = act

  cat_spec = pl.BlockSpec((_RB, 3 * D), lambda i: (i, 0))
  cat_shape = jax.ShapeDtypeStruct((N, 3 * D), jnp.float32)
  if last:
    out_specs, out_shape = cat_spec, cat_shape
  else:
    out_specs = (pl.BlockSpec((_RB, D), lambda i: (i, 0)), cat_spec)
    out_shape = (jax.ShapeDtypeStruct((N, D), jnp.float32), cat_shape)

  return pl.pallas_call(
      body,
      grid=(N // _RB,),
      in_specs=in_specs,
      out_specs=out_specs,
      out_shape=out_shape,
  )(*args)


def kernel(x, pos, edge_index, batch, Wp, bp, W0, b0, W1, b1, W2, b2, W3, b3):
  del batch, W3, b3
  ei = edge_index.reshape(2, NTILES, NCH, CH)
  z = jnp.zeros((NPAD, D), jnp.float32)
  zd = jnp.zeros((NPAD, DW), jnp.float32)

  wx = Wp[POS_DIM:]
  wp = Wp[:POS_DIM]

  h0 = _proj(x, pos, wx, wp, bp.reshape(1, D))
  agg1, deg = _sc_agg_deg(h0, ei, z, zd)
  h1, hcat = _layer(0, h0, agg1, deg, W0, b0.reshape(1, D))
  agg2 = _sc_agg(h1, ei, z)
  h2, hcat = _layer(1, h1, agg2, deg, W1, b1.reshape(1, D), hcat)
  agg3 = _sc_agg(h2, ei, z)
  hcat = _layer(2, h2, agg3, deg, W2, b2.reshape(1, D), hcat)
  return hcat


# speedup vs baseline: 5.1392x; 1.0003x over previous
import jax
import jax.numpy as jnp
from jax import lax
from jax.experimental import pallas as pl
from jax.experimental.pallas import tpu as pltpu
from jax.experimental.pallas import tpu_sc as plsc

N = 10000
E = 320000
D = 64
NODE_DIM = 128
POS_DIM = 3

NUM_CORES = 2
NUM_SUBCORES = 16
NTILES = NUM_CORES * NUM_SUBCORES
CH = 125
NCH = 80
NPAD = 10112
ZROWS = NPAD // NUM_SUBCORES
LAST_ROWS = N - 15 * ZROWS
DW = 16

_MESH = plsc.VectorSubcoreMesh(
    core_axis_name="c", subcore_axis_name="s",
    num_cores=NUM_CORES, num_subcores=NUM_SUBCORES)
_SC_PARAMS = pltpu.CompilerParams(use_tc_tiling_on_sc=False)


def _row_split(s, agg_sh, agg_out, c):
  @pl.when(s < NUM_SUBCORES - 1)
  def _():
    osl = pl.ds(s * ZROWS, ZROWS)
    pltpu.sync_copy(agg_sh.at[osl], agg_out.at[c, osl])

  @pl.when(s == NUM_SUBCORES - 1)
  def _():
    osl = pl.ds(s * ZROWS, LAST_ROWS)
    pltpu.sync_copy(agg_sh.at[osl], agg_out.at[c, osl])


NB = 8


def _sc_body(nb, with_deg, *refs):
  if with_deg:
    (h_hbm, ei_hbm, z_hbm, zd_hbm, agg_out, deg_out, src_v, dst_v), rest = (
        refs[:8], refs[8:])
  else:
    (h_hbm, ei_hbm, z_hbm, agg_out, src_v, dst_v), rest = refs[:6], refs[6:]
  rows = rest[:nb]
  rest = rest[nb:]
  if with_deg:
    ones_v, deg_sh = rest[0], rest[2]
    agg_sh = rest[1]
    rest = rest[3:]
  else:
    agg_sh = rest[0]
    rest = rest[1:]
  sg = rest[:nb]
  ss = rest[nb:]
  pf = nb // 2
  c = lax.axis_index("c")
  s = lax.axis_index("s")
  wid = s * NUM_CORES + c

  zsl = pl.ds(s * ZROWS, ZROWS)
  pltpu.sync_copy(z_hbm.at[zsl], agg_sh.at[zsl])
  if with_deg:
    pltpu.sync_copy(zd_hbm.at[zsl], deg_sh.at[zsl])

    def fill(i, carry):
      ones_v[i] = jnp.ones((DW,), jnp.float32)
      return carry
    lax.fori_loop(0, CH, fill, 0)
  plsc.subcore_barrier()

  pltpu.sync_copy(ei_hbm.at[0, wid], src_v)
  pltpu.sync_copy(ei_hbm.at[1, wid], dst_v)

  def g(i, b):
    pltpu.async_copy(h_hbm.at[src_v.at[i]], rows[b], sg[b])

  def wg(b):
    pltpu.make_async_copy(h_hbm.at[src_v.at[0]], rows[b], sg[b]).wait()

  def sc(i, b):
    pltpu.async_copy(rows[b], agg_sh.at[dst_v.at[i]], ss[b], add=True)
    if with_deg:
      pltpu.sync_copy(ones_v, deg_sh.at[dst_v.at[i]], add=True)

  def ws(b):
    pltpu.make_async_copy(rows[b], agg_sh.at[dst_v.at[0]], ss[b]).wait()

  for i in range(pf):
    g(i, i)
  for c0 in range(pf):
    wg(c0 % nb); sc(c0, c0 % nb); g(c0 + pf, (c0 + pf) % nb)
  for c0 in range(pf, nb):
    wg(c0 % nb); sc(c0, c0 % nb); ws((c0 + pf) % nb); g(c0 + pf, (c0 + pf) % nb)

  def ring(q, carry):
    c0 = nb * q
    for r in range(nb):
      tb = (r + pf) % nb
      wg(r); sc(c0 + r, r); ws(tb)
      g(c0 + r + pf, tb)
    return carry
  lax.fori_loop(1, NCH // nb - 1, ring, 0)

  for r in range(NCH % nb + nb):
    c0 = (NCH // nb - 1) * nb + r
    b = c0 % nb
    wg(b); sc(c0, b)
    if c0 + pf < NCH:
      ws((c0 + pf) % nb); g(c0 + pf, (c0 + pf) % nb)
  for b in range(nb):
    ws(b)

  plsc.subcore_barrier()
  _row_split(s, agg_sh, agg_out, c)
  if with_deg:
    _row_split(s, deg_sh, deg_out, c)


def _make_sc(nb, with_deg):
  import functools
  out_type = [jax.ShapeDtypeStruct((NUM_CORES, N, D), jnp.float32)]
  scratch = [pltpu.VMEM((NCH, CH), jnp.int32)] * 2
  scratch += [pltpu.VMEM((CH, D), jnp.float32)] * nb
  if with_deg:
    out_type.append(jax.ShapeDtypeStruct((NUM_CORES, N, DW), jnp.float32))
    scratch.append(pltpu.VMEM((CH, DW), jnp.float32))
  scratch.append(pltpu.VMEM_SHARED((NPAD, D), jnp.float32))
  if with_deg:
    scratch.append(pltpu.VMEM_SHARED((NPAD, DW), jnp.float32))
  scratch += [pltpu.SemaphoreType.DMA] * (2 * nb)
  return pl.kernel(
      functools.partial(_sc_body, nb, with_deg),
      out_type=tuple(out_type) if with_deg else out_type[0],
      mesh=_MESH,
      compiler_params=_SC_PARAMS,
      scratch_types=scratch,
  )


_sc_agg = _make_sc(NB, False)
_sc_agg_deg = _make_sc(6, True)


def _proj_body(x_ref, pos_ref, wx_ref, wp_ref, b_ref, o_ref):
  acc = lax.dot_general(
      x_ref[...], wx_ref[...], (((1,), (0,)), ((), ())),
      precision=lax.Precision.HIGHEST, preferred_element_type=jnp.float32)
  acc += lax.dot_general(
      pos_ref[...], wp_ref[...], (((1,), (0,)), ((), ())),
      precision=lax.Precision.HIGHEST, preferred_element_type=jnp.float32)
  o_ref[...] = acc + b_ref[...]


def _act(h_ref, agg_ref, deg_ref, w_ref, b_ref):
  agg = agg_ref[0] + agg_ref[1]
  deg = jnp.maximum(deg_ref[0, :, 0:1] + deg_ref[1, :, 0:1], 1.0)
  m = h_ref[...] + agg / deg
  out = lax.dot_general(
      m, w_ref[...], (((1,), (0,)), ((), ())),
      precision=lax.Precision.HIGHEST, preferred_element_type=jnp.float32)
  out = out + b_ref[...]
  return jnp.where(out >= 0.0, out, 0.01 * out)


_RB = 2000


def _proj(x, pos, wx, wp, b):
  return pl.pallas_call(
      _proj_body,
      grid=(N // _RB,),
      in_specs=[
          pl.BlockSpec((_RB, NODE_DIM), lambda i: (i, 0)),
          pl.BlockSpec((_RB, POS_DIM), lambda i: (i, 0)),
          pl.BlockSpec((NODE_DIM, D), lambda i: (0, 0)),
          pl.BlockSpec((POS_DIM, D), lambda i: (0, 0)),
          pl.BlockSpec((1, D), lambda i: (0, 0)),
      ],
      out_specs=pl.BlockSpec((_RB, D), lambda i: (i, 0)),
      out_shape=jax.ShapeDtypeStruct((N, D), jnp.float32),
  )(x, pos, wx, wp, b)


def _layer(l, h, agg, deg, w, b, hcat=None):
  in_specs = [
      pl.BlockSpec((_RB, D), lambda i: (i, 0)),
      pl.BlockSpec((NUM_CORES, _RB, D), lambda i: (0, i, 0)),
      pl.BlockSpec((NUM_CORES, _RB, DW), lambda i: (0, i, 0)),
      pl.BlockSpec((D, D), lambda i: (0, 0)),
      pl.BlockSpec((1, D), lambda i: (0, 0)),
  ]
  args = [h, agg, deg, w, b]
  if l > 0:
    in_specs.append(pl.BlockSpec((_RB, 3 * D), lambda i: (i, 0)))
    args.append(hcat)
  last = l == 2

  def body(h_ref, agg_ref, deg_ref, w_ref, b_ref, *rest):
    act = _act(h_ref, agg_ref, deg_ref, w_ref, b_ref)
    if l == 0:
      o_ref, cat_ref = rest
      cat_ref[...] = jnp.concatenate(
          [act, jnp.zeros((act.shape[0], 2 * D), jnp.float32)], axis=1)
    elif l == 1:
      cat_in, o_ref, cat_ref = rest
      cat_ref[...] = jnp.concatenate(
          [cat_in[:, :D], act, jnp.zeros((act.shape[0], D), jnp.float32)],
          axis=1)
    else:
      cat_in, cat_ref = rest
      cat_ref[...] = jnp.concatenate([cat_in[:, :2 * D], act], axis=1)
      return
    o_ref[...] = act

  cat_spec = pl.BlockSpec((_RB, 3 * D), lambda i: (i, 0))
  cat_shape = jax.ShapeDtypeStruct((N, 3 * D), jnp.float32)
  if last:
    out_specs, out_shape = cat_spec, cat_shape
  else:
    out_specs = (pl.BlockSpec((_RB, D), lambda i: (i, 0)), cat_spec)
    out_shape = (jax.ShapeDtypeStruct((N, D), jnp.float32), cat_shape)

  return pl.pallas_call(
      body,
      grid=(N // _RB,),
      in_specs=in_specs,
      out_specs=out_specs,
      out_shape=out_shape,
  )(*args)


def kernel(x, pos, edge_index, batch, Wp, bp, W0, b0, W1, b1, W2, b2, W3, b3):
  del batch, W3, b3
  ei = edge_index.reshape(2, NTILES, NCH, CH)
  z = jnp.zeros((NPAD, D), jnp.float32)
  zd = jnp.zeros((NPAD, DW), jnp.float32)

  wx = Wp[POS_DIM:]
  wp = Wp[:POS_DIM]

  h0 = _proj(x, pos, wx, wp, bp.reshape(1, D))
  agg1, deg = _sc_agg_deg(h0, ei, z, zd)
  h1, hcat = _layer(0, h0, agg1, deg, W0, b0.reshape(1, D))
  agg2 = _sc_agg(h1, ei, z)
  h2, hcat = _layer(1, h1, agg2, deg, W1, b1.reshape(1, D), hcat)
  agg3 = _sc_agg(h2, ei, z)
  hcat = _layer(2, h2, agg3, deg, W2, b2.reshape(1, D), hcat)
  return hcat
